# Initial kernel scaffold; baseline (speedup 1.0000x reference)
import jax
import jax.numpy as jnp
from jax.experimental import pallas as pl


def kernel(x, ei, batch, eps, w1, b1, g1, be1, w2, b2, g2, be2, cw1, cb1, cw2, cb2):
    raise NotImplementedError("write your pallas kernel here")


# speedup vs baseline: 2.5596x; 2.5596x over previous
import functools

import jax
import jax.numpy as jnp
from jax import lax
from jax.experimental import pallas as pl
from jax.experimental.pallas import tpu as pltpu
from jax.experimental.pallas import tpu_sc as plsc

N = 10000
E = 320000
D = 128
H = 128
OUT = 64
G = 64
L = 3

NC = 2
NS = 16
NW = NC * NS
CHUNK = 128

EDGE_CHUNKS = 80
E_PAD = NW * EDGE_CHUNKS * CHUNK
ACC_ROWS = 10112
ROWS_PER_TILE = ACC_ROWS // NS

POOL_CHUNKS = 3
N_POOL_PAD = NW * POOL_CHUNKS * CHUNK
POOL_ACC_ROWS = 128
POOL_ROWS_PER_TILE = POOL_ACC_ROWS // NS


def _make_scatter_add(chunks_per_tile, acc_rows):
    rows_per_tile = acc_rows // NS
    mesh = plsc.VectorSubcoreMesh(core_axis_name="c", subcore_axis_name="s")

    @functools.partial(
        pl.kernel,
        mesh=mesh,
        out_type=jax.ShapeDtypeStruct((NC, acc_rows, D), jnp.float32),
        scratch_types=[
            pltpu.VMEM((chunks_per_tile, CHUNK), jnp.int32),
            pltpu.VMEM((chunks_per_tile, CHUNK), jnp.int32),
            pltpu.VMEM((CHUNK, D), jnp.float32),
            pltpu.VMEM_SHARED((acc_rows, D), jnp.float32),
            pltpu.SemaphoreType.DMA,
        ],
    )
    def k(table_hbm, src_hbm, dst_hbm, out_hbm, idx_s, idx_d, rows, acc, sem):
        cid = lax.axis_index("c")
        sid = lax.axis_index("s")
        wid = sid * NC + cid

        def zero_row(i, carry):
            for j in range(D // 16):
                rows[i, pl.ds(j * 16, 16)] = jnp.zeros((16,), jnp.float32)
            return carry

        lax.fori_loop(0, CHUNK, zero_row, 0)
        base = sid * rows_per_tile
        full = rows_per_tile // CHUNK
        rem = rows_per_tile % CHUNK
        for c in range(full):
            pltpu.sync_copy(rows, acc.at[pl.ds(base + c * CHUNK, CHUNK)])
        if rem:
            pltpu.sync_copy(rows.at[pl.ds(0, rem)],
                            acc.at[pl.ds(base + full * CHUNK, rem)])
        plsc.subcore_barrier()

        pltpu.sync_copy(src_hbm.at[wid], idx_s)
        pltpu.sync_copy(dst_hbm.at[wid], idx_d)

        def step(j, carry):
            pltpu.async_copy(table_hbm.at[idx_s.at[j]], rows, sem).wait()
            pltpu.sync_copy(rows, acc.at[idx_d.at[j]], add=True)
            return carry

        lax.fori_loop(0, chunks_per_tile, step, 0)
        plsc.subcore_barrier()

        pltpu.sync_copy(acc.at[pl.ds(base, rows_per_tile)],
                        out_hbm.at[cid, pl.ds(base, rows_per_tile)])

    return k


BLK = 400


def _mlp_body(scale_ref, h_ref, p_ref, w1_ref, b1_ref, w2_ref, b2_ref, o_ref):
    z = h_ref[...] * scale_ref[...] + p_ref[0] + p_ref[1]
    a = jnp.dot(z, w1_ref[...], preferred_element_type=jnp.float32) + b1_ref[...]
    a = jnp.maximum(a, 0.0)
    o = jnp.dot(a, w2_ref[...], preferred_element_type=jnp.float32) + b2_ref[...]
    o_ref[...] = jnp.maximum(o, 0.0)


def _mlp(scale, h, parts, w1f, b1f, w2f, b2f):
    return pl.pallas_call(
        _mlp_body,
        grid=(N // BLK,),
        in_specs=[
            pl.BlockSpec((1, H), lambda i: (0, 0)),
            pl.BlockSpec((BLK, D), lambda i: (i, 0)),
            pl.BlockSpec((NC, BLK, D), lambda i: (0, i, 0)),
            pl.BlockSpec((D, H), lambda i: (0, 0)),
            pl.BlockSpec((1, H), lambda i: (0, 0)),
            pl.BlockSpec((H, H), lambda i: (0, 0)),
            pl.BlockSpec((1, H), lambda i: (0, 0)),
        ],
        out_specs=pl.BlockSpec((BLK, H), lambda i: (i, 0)),
        out_shape=jax.ShapeDtypeStruct((N, H), jnp.float32),
    )(scale, h, parts, w1f, b1f, w2f, b2f)


def _head_body(p_ref, cw1_ref, cb1_ref, cw2_ref, cb2_ref, o_ref):
    pooled = p_ref[0] + p_ref[1]
    a = jnp.dot(pooled, cw1_ref[...], preferred_element_type=jnp.float32) + cb1_ref[...]
    a = jnp.maximum(a, 0.0)
    o_ref[...] = jnp.dot(a, cw2_ref[...], preferred_element_type=jnp.float32) + cb2_ref[...]


def _head(parts, cw1, cb1, cw2, cb2):
    return pl.pallas_call(
        _head_body,
        grid=(1,),
        in_specs=[
            pl.BlockSpec((NC, G, H), lambda i: (0, 0, 0)),
            pl.BlockSpec((H, H), lambda i: (0, 0)),
            pl.BlockSpec((1, H), lambda i: (0, 0)),
            pl.BlockSpec((H, OUT), lambda i: (0, 0)),
            pl.BlockSpec((1, OUT), lambda i: (0, 0)),
        ],
        out_specs=pl.BlockSpec((G, OUT), lambda i: (0, 0)),
        out_shape=jax.ShapeDtypeStruct((G, OUT), jnp.float32),
    )(parts, cw1, cb1, cw2, cb2)


def kernel(x, ei, batch, eps, w1, b1, g1, be1, w2, b2, g2, be2, cw1, cb1, cw2, cb2):
    s1 = g1 / jnp.sqrt(1.0 + 1e-5)
    w1f = w1 * s1[:, None, :]
    b1f = (b1 * s1 + be1)[:, None, :]
    s2 = g2 / jnp.sqrt(1.0 + 1e-5)
    w2f = w2 * s2[:, None, :]
    b2f = (b2 * s2 + be2)[:, None, :]

    pad_e = E_PAD - E
    src = jnp.concatenate(
        [ei[0], jnp.zeros((pad_e,), jnp.int32)]).reshape(NW, EDGE_CHUNKS, CHUNK)
    dst = jnp.concatenate(
        [ei[1], jnp.full((pad_e,), ACC_ROWS - 1, jnp.int32)]
    ).reshape(NW, EDGE_CHUNKS, CHUNK)

    pad_n = N_POOL_PAD - N
    psrc = jnp.concatenate(
        [jnp.arange(N, dtype=jnp.int32),
         jnp.zeros((pad_n,), jnp.int32)]).reshape(NW, POOL_CHUNKS, CHUNK)
    pdst = jnp.concatenate(
        [batch, jnp.full((pad_n,), POOL_ACC_ROWS - 1, jnp.int32)]
    ).reshape(NW, POOL_CHUNKS, CHUNK)

    edge_scatter = _make_scatter_add(EDGE_CHUNKS, ACC_ROWS)
    pool_scatter = _make_scatter_add(POOL_CHUNKS, POOL_ACC_ROWS)

    h = x
    for i in range(L):
        parts = edge_scatter(h, src, dst)
        scale = jnp.full((1, H), 1.0 + eps[i], jnp.float32)
        h = _mlp(scale, h, parts, w1f[i], b1f[i], w2f[i], b2f[i])

    pparts = pool_scatter(h, psrc, pdst)[:, :G, :]
    return _head(pparts, cw1, cb1[None], cw2, cb2[None])


# speedup vs baseline: 2.8473x; 1.1124x over previous
import functools

import jax
import jax.numpy as jnp
from jax import lax
from jax.experimental import pallas as pl
from jax.experimental.pallas import tpu as pltpu
from jax.experimental.pallas import tpu_sc as plsc

N = 10000
E = 320000
D = 128
H = 128
OUT = 64
G = 64
L = 3

NC = 2
NS = 16
NW = NC * NS
CHUNK = 128

EDGE_CHUNKS = 80
E_PAD = NW * EDGE_CHUNKS * CHUNK
ACC_ROWS = 10112
ROWS_PER_TILE = ACC_ROWS // NS

POOL_CHUNKS = 3
N_POOL_PAD = NW * POOL_CHUNKS * CHUNK
POOL_ACC_ROWS = 128
POOL_ROWS_PER_TILE = POOL_ACC_ROWS // NS


def _make_scatter_add(chunks_per_tile, acc_rows, phases):
    rows_per_tile = acc_rows // NS
    cpp = chunks_per_tile // phases
    nbuf = 2 if cpp % 2 == 0 else cpp
    n_outer = cpp // nbuf
    mesh = plsc.VectorSubcoreMesh(core_axis_name="c", subcore_axis_name="s")

    @functools.partial(
        pl.kernel,
        mesh=mesh,
        out_type=jax.ShapeDtypeStruct((NC, acc_rows, D), jnp.float32),
        scratch_types=[
            pltpu.VMEM((cpp, CHUNK), jnp.int32),
            pltpu.VMEM((cpp, CHUNK), jnp.int32),
            pltpu.VMEM((nbuf, CHUNK, D), jnp.float32),
            pltpu.VMEM_SHARED((acc_rows, D), jnp.float32),
            pltpu.SemaphoreType.DMA((nbuf,)),
            pltpu.SemaphoreType.DMA((nbuf,)),
        ],
    )
    def k(table_hbm, src_hbm, dst_hbm, out_hbm, idx_s, idx_d, rows, acc,
          gsem, ssem):
        cid = lax.axis_index("c")
        sid = lax.axis_index("s")
        wid = sid * NC + cid

        def zero_row(i, carry):
            for j in range(D // 16):
                rows[0, i, pl.ds(j * 16, 16)] = jnp.zeros((16,), jnp.float32)
            return carry

        lax.fori_loop(0, CHUNK, zero_row, 0)
        base = sid * rows_per_tile
        full = rows_per_tile // CHUNK
        rem = rows_per_tile % CHUNK
        for c in range(full):
            pltpu.sync_copy(rows.at[0], acc.at[pl.ds(base + c * CHUNK, CHUNK)])
        if rem:
            pltpu.sync_copy(rows.at[0, pl.ds(0, rem)],
                            acc.at[pl.ds(base + full * CHUNK, rem)])
        plsc.subcore_barrier()

        for p in range(phases):
            pltpu.sync_copy(src_hbm.at[wid, pl.ds(p * cpp, cpp)], idx_s)
            pltpu.sync_copy(dst_hbm.at[wid, pl.ds(p * cpp, cpp)], idx_d)
            for b in range(nbuf):
                pltpu.async_copy(table_hbm.at[idx_s.at[b]], rows.at[b],
                                 gsem.at[b])

            def outer(jj, carry):
                c0 = jj * nbuf
                for b in range(nbuf):
                    c = c0 + b
                    pltpu.make_async_copy(table_hbm.at[idx_s.at[c]],
                                          rows.at[b], gsem.at[b]).wait()
                    pltpu.async_copy(rows.at[b], acc.at[idx_d.at[c]],
                                     ssem.at[b], add=True)
                    pltpu.make_async_copy(rows.at[b], acc.at[idx_d.at[c]],
                                          ssem.at[b]).wait()

                    @pl.when(jj < n_outer - 1)
                    def _():
                        pltpu.async_copy(table_hbm.at[idx_s.at[c + nbuf]],
                                         rows.at[b], gsem.at[b])
                return carry

            lax.fori_loop(0, n_outer, outer, 0)
        plsc.subcore_barrier()

        pltpu.sync_copy(acc.at[pl.ds(base, rows_per_tile)],
                        out_hbm.at[cid, pl.ds(base, rows_per_tile)])

    return k


BLK = 400


def _mlp_body(scale_ref, h_ref, p_ref, w1_ref, b1_ref, w2_ref, b2_ref, o_ref):
    z = h_ref[...] * scale_ref[...] + p_ref[0] + p_ref[1]
    a = jnp.dot(z, w1_ref[...], preferred_element_type=jnp.float32) + b1_ref[...]
    a = jnp.maximum(a, 0.0)
    o = jnp.dot(a, w2_ref[...], preferred_element_type=jnp.float32) + b2_ref[...]
    o_ref[...] = jnp.maximum(o, 0.0)


def _mlp(scale, h, parts, w1f, b1f, w2f, b2f):
    return pl.pallas_call(
        _mlp_body,
        grid=(N // BLK,),
        in_specs=[
            pl.BlockSpec((1, H), lambda i: (0, 0)),
            pl.BlockSpec((BLK, D), lambda i: (i, 0)),
            pl.BlockSpec((NC, BLK, D), lambda i: (0, i, 0)),
            pl.BlockSpec((D, H), lambda i: (0, 0)),
            pl.BlockSpec((1, H), lambda i: (0, 0)),
            pl.BlockSpec((H, H), lambda i: (0, 0)),
            pl.BlockSpec((1, H), lambda i: (0, 0)),
        ],
        out_specs=pl.BlockSpec((BLK, H), lambda i: (i, 0)),
        out_shape=jax.ShapeDtypeStruct((N, H), jnp.float32),
    )(scale, h, parts, w1f, b1f, w2f, b2f)


def _head_body(p_ref, cw1_ref, cb1_ref, cw2_ref, cb2_ref, o_ref):
    pooled = p_ref[0] + p_ref[1]
    a = jnp.dot(pooled, cw1_ref[...], preferred_element_type=jnp.float32) + cb1_ref[...]
    a = jnp.maximum(a, 0.0)
    o_ref[...] = jnp.dot(a, cw2_ref[...], preferred_element_type=jnp.float32) + cb2_ref[...]


def _head(parts, cw1, cb1, cw2, cb2):
    return pl.pallas_call(
        _head_body,
        grid=(1,),
        in_specs=[
            pl.BlockSpec((NC, G, H), lambda i: (0, 0, 0)),
            pl.BlockSpec((H, H), lambda i: (0, 0)),
            pl.BlockSpec((1, H), lambda i: (0, 0)),
            pl.BlockSpec((H, OUT), lambda i: (0, 0)),
            pl.BlockSpec((1, OUT), lambda i: (0, 0)),
        ],
        out_specs=pl.BlockSpec((G, OUT), lambda i: (0, 0)),
        out_shape=jax.ShapeDtypeStruct((G, OUT), jnp.float32),
    )(parts, cw1, cb1, cw2, cb2)


def kernel(x, ei, batch, eps, w1, b1, g1, be1, w2, b2, g2, be2, cw1, cb1, cw2, cb2):
    s1 = g1 / jnp.sqrt(1.0 + 1e-5)
    w1f = w1 * s1[:, None, :]
    b1f = (b1 * s1 + be1)[:, None, :]
    s2 = g2 / jnp.sqrt(1.0 + 1e-5)
    w2f = w2 * s2[:, None, :]
    b2f = (b2 * s2 + be2)[:, None, :]

    pad_e = E_PAD - E
    src = jnp.concatenate(
        [ei[0], jnp.zeros((pad_e,), jnp.int32)]).reshape(NW, EDGE_CHUNKS, CHUNK)
    dst = jnp.concatenate(
        [ei[1], jnp.full((pad_e,), ACC_ROWS - 1, jnp.int32)]
    ).reshape(NW, EDGE_CHUNKS, CHUNK)

    pad_n = N_POOL_PAD - N
    psrc = jnp.concatenate(
        [jnp.arange(N, dtype=jnp.int32),
         jnp.zeros((pad_n,), jnp.int32)]).reshape(NW, POOL_CHUNKS, CHUNK)
    pdst = jnp.concatenate(
        [batch, jnp.full((pad_n,), POOL_ACC_ROWS - 1, jnp.int32)]
    ).reshape(NW, POOL_CHUNKS, CHUNK)

    edge_scatter = _make_scatter_add(EDGE_CHUNKS, ACC_ROWS, phases=2)
    pool_scatter = _make_scatter_add(POOL_CHUNKS, POOL_ACC_ROWS, phases=1)

    h = x
    for i in range(L):
        parts = edge_scatter(h, src, dst)
        scale = jnp.full((1, H), 1.0 + eps[i], jnp.float32)
        h = _mlp(scale, h, parts, w1f[i], b1f[i], w2f[i], b2f[i])

    pparts = pool_scatter(h, psrc, pdst)[:, :G, :]
    return _head(pparts, cw1, cb1[None], cw2, cb2[None])


# speedup vs baseline: 2.8769x; 1.0104x over previous
import functools

import jax
import jax.numpy as jnp
from jax import lax
from jax.experimental import pallas as pl
from jax.experimental.pallas import tpu as pltpu
from jax.experimental.pallas import tpu_sc as plsc

N = 10000
E = 320000
D = 128
H = 128
OUT = 64
G = 64
L = 3

NC = 2
NS = 16
NW = NC * NS
CHUNK = 128

EDGE_CHUNKS = 80
E_PAD = NW * EDGE_CHUNKS * CHUNK
ACC_ROWS = 10112
ROWS_PER_TILE = ACC_ROWS // NS

def _make_scatter_add(chunks_per_tile, acc_rows, phases):
    rows_per_tile = acc_rows // NS
    cpp = chunks_per_tile // phases
    nbuf = 2 if cpp % 2 == 0 else cpp
    n_outer = cpp // nbuf
    mesh = plsc.VectorSubcoreMesh(core_axis_name="c", subcore_axis_name="s")

    @functools.partial(
        pl.kernel,
        mesh=mesh,
        out_type=jax.ShapeDtypeStruct((NC, acc_rows, D), jnp.float32),
        scratch_types=[
            pltpu.VMEM((cpp, CHUNK), jnp.int32),
            pltpu.VMEM((cpp, CHUNK), jnp.int32),
            pltpu.VMEM((nbuf, CHUNK, D), jnp.float32),
            pltpu.VMEM_SHARED((acc_rows, D), jnp.float32),
            pltpu.SemaphoreType.DMA((nbuf,)),
            pltpu.SemaphoreType.DMA((nbuf,)),
            pltpu.SemaphoreType.DMA,
        ],
    )
    def k(table_hbm, src_hbm, dst_hbm, zeros_hbm, out_hbm, idx_s, idx_d,
          rows, acc, gsem, ssem, zsem):
        cid = lax.axis_index("c")
        sid = lax.axis_index("s")
        wid = sid * NC + cid

        base = sid * rows_per_tile
        zcopy = pltpu.async_copy(zeros_hbm.at[pl.ds(0, rows_per_tile)],
                                 acc.at[pl.ds(base, rows_per_tile)], zsem)
        zcopy.wait()
        plsc.subcore_barrier()

        for p in range(phases):
            pltpu.sync_copy(src_hbm.at[wid, pl.ds(p * cpp, cpp)], idx_s)
            pltpu.sync_copy(dst_hbm.at[wid, pl.ds(p * cpp, cpp)], idx_d)
            for b in range(nbuf):
                pltpu.async_copy(table_hbm.at[idx_s.at[b]], rows.at[b],
                                 gsem.at[b])

            def outer(jj, carry):
                c0 = jj * nbuf
                for b in range(nbuf):
                    c = c0 + b
                    pltpu.make_async_copy(table_hbm.at[idx_s.at[c]],
                                          rows.at[b], gsem.at[b]).wait()
                    pltpu.async_copy(rows.at[b], acc.at[idx_d.at[c]],
                                     ssem.at[b], add=True)
                    pltpu.make_async_copy(rows.at[b], acc.at[idx_d.at[c]],
                                          ssem.at[b]).wait()

                    @pl.when(jj < n_outer - 1)
                    def _():
                        pltpu.async_copy(table_hbm.at[idx_s.at[c + nbuf]],
                                         rows.at[b], gsem.at[b])
                return carry

            lax.fori_loop(0, n_outer, outer, 0)
        plsc.subcore_barrier()

        pltpu.sync_copy(acc.at[pl.ds(base, rows_per_tile)],
                        out_hbm.at[cid, pl.ds(base, rows_per_tile)])

    return k


BLK = 400


def _mlp_body(scale_ref, h_ref, p_ref, w1_ref, b1_ref, w2_ref, b2_ref, o_ref):
    z = h_ref[...] * scale_ref[...] + p_ref[0] + p_ref[1]
    a = jnp.dot(z, w1_ref[...], preferred_element_type=jnp.float32) + b1_ref[...]
    a = jnp.maximum(a, 0.0)
    o = jnp.dot(a, w2_ref[...], preferred_element_type=jnp.float32) + b2_ref[...]
    o_ref[...] = jnp.maximum(o, 0.0)


def _mlp(scale, h, parts, w1f, b1f, w2f, b2f):
    return pl.pallas_call(
        _mlp_body,
        grid=(N // BLK,),
        in_specs=[
            pl.BlockSpec((1, H), lambda i: (0, 0)),
            pl.BlockSpec((BLK, D), lambda i: (i, 0)),
            pl.BlockSpec((NC, BLK, D), lambda i: (0, i, 0)),
            pl.BlockSpec((D, H), lambda i: (0, 0)),
            pl.BlockSpec((1, H), lambda i: (0, 0)),
            pl.BlockSpec((H, H), lambda i: (0, 0)),
            pl.BlockSpec((1, H), lambda i: (0, 0)),
        ],
        out_specs=pl.BlockSpec((BLK, H), lambda i: (i, 0)),
        out_shape=jax.ShapeDtypeStruct((N, H), jnp.float32),
    )(scale, h, parts, w1f, b1f, w2f, b2f)


def _head_body(h_ref, oh_ref, cw1_ref, cb1_ref, cw2_ref, cb2_ref, o_ref,
               acc_ref):
    i = pl.program_id(0)

    @pl.when(i == 0)
    def _():
        acc_ref[...] = jnp.zeros_like(acc_ref)

    acc_ref[...] += lax.dot_general(
        oh_ref[...], h_ref[...], (((0,), (0,)), ((), ())),
        preferred_element_type=jnp.float32)

    @pl.when(i == pl.num_programs(0) - 1)
    def _():
        p = acc_ref[...]
        a = jnp.dot(p, cw1_ref[...], preferred_element_type=jnp.float32) + cb1_ref[...]
        a = jnp.maximum(a, 0.0)
        o_ref[...] = jnp.dot(a, cw2_ref[...], preferred_element_type=jnp.float32) + cb2_ref[...]


def _head(h, onehot, cw1, cb1, cw2, cb2):
    return pl.pallas_call(
        _head_body,
        grid=(N // BLK,),
        in_specs=[
            pl.BlockSpec((BLK, D), lambda i: (i, 0)),
            pl.BlockSpec((BLK, G), lambda i: (i, 0)),
            pl.BlockSpec((H, H), lambda i: (0, 0)),
            pl.BlockSpec((1, H), lambda i: (0, 0)),
            pl.BlockSpec((H, OUT), lambda i: (0, 0)),
            pl.BlockSpec((1, OUT), lambda i: (0, 0)),
        ],
        out_specs=pl.BlockSpec((G, OUT), lambda i: (0, 0)),
        out_shape=jax.ShapeDtypeStruct((G, OUT), jnp.float32),
        scratch_shapes=[pltpu.VMEM((G, H), jnp.float32)],
    )(h, onehot, cw1, cb1, cw2, cb2)


def kernel(x, ei, batch, eps, w1, b1, g1, be1, w2, b2, g2, be2, cw1, cb1, cw2, cb2):
    s1 = g1 / jnp.sqrt(1.0 + 1e-5)
    w1f = w1 * s1[:, None, :]
    b1f = (b1 * s1 + be1)[:, None, :]
    s2 = g2 / jnp.sqrt(1.0 + 1e-5)
    w2f = w2 * s2[:, None, :]
    b2f = (b2 * s2 + be2)[:, None, :]

    pad_e = E_PAD - E
    src = jnp.concatenate(
        [ei[0], jnp.zeros((pad_e,), jnp.int32)]).reshape(NW, EDGE_CHUNKS, CHUNK)
    dst = jnp.concatenate(
        [ei[1], jnp.full((pad_e,), ACC_ROWS - 1, jnp.int32)]
    ).reshape(NW, EDGE_CHUNKS, CHUNK)

    onehot = (batch[:, None] == jnp.arange(G, dtype=jnp.int32)[None, :]
              ).astype(jnp.float32)

    zeros = jnp.zeros((ROWS_PER_TILE, D), jnp.float32)
    edge_scatter = _make_scatter_add(EDGE_CHUNKS, ACC_ROWS, phases=2)

    h = x
    for i in range(L):
        parts = edge_scatter(h, src, dst, zeros)
        scale = jnp.full((1, H), 1.0 + eps[i], jnp.float32)
        h = _mlp(scale, h, parts, w1f[i], b1f[i], w2f[i], b2f[i])

    return _head(h, onehot, cw1, cb1[None], cw2, cb2[None])


# speedup vs baseline: 2.8871x; 1.0035x over previous
import functools

import jax
import jax.numpy as jnp
from jax import lax
from jax.experimental import pallas as pl
from jax.experimental.pallas import tpu as pltpu
from jax.experimental.pallas import tpu_sc as plsc

N = 10000
E = 320000
D = 128
H = 128
OUT = 64
G = 64
L = 3

NC = 2
NS = 16
NW = NC * NS
CHUNK = 128

EDGE_CHUNKS = 80
E_PAD = NW * EDGE_CHUNKS * CHUNK
ACC_ROWS = 10112
ROWS_PER_TILE = ACC_ROWS // NS

def _make_scatter_add(chunks_per_tile, acc_rows, phases):
    rows_per_tile = acc_rows // NS
    cpp = chunks_per_tile // phases
    nbuf = 2 if cpp % 2 == 0 else cpp
    n_outer = cpp // nbuf
    mesh = plsc.VectorSubcoreMesh(core_axis_name="c", subcore_axis_name="s")

    @functools.partial(
        pl.kernel,
        mesh=mesh,
        out_type=jax.ShapeDtypeStruct((NC, acc_rows, D), jnp.float32),
        scratch_types=[
            pltpu.VMEM((cpp, CHUNK), jnp.int32),
            pltpu.VMEM((cpp, CHUNK), jnp.int32),
            pltpu.VMEM((nbuf, CHUNK, D), jnp.float32),
            pltpu.VMEM_SHARED((acc_rows, D), jnp.float32),
            pltpu.SemaphoreType.DMA((nbuf,)),
            pltpu.SemaphoreType.DMA((nbuf,)),
            pltpu.SemaphoreType.DMA,
        ],
    )
    def k(table_hbm, src_hbm, dst_hbm, zeros_hbm, out_hbm, idx_s, idx_d,
          rows, acc, gsem, ssem, zsem):
        cid = lax.axis_index("c")
        sid = lax.axis_index("s")
        wid = sid * NC + cid

        base = sid * rows_per_tile
        zcopy = pltpu.async_copy(zeros_hbm.at[pl.ds(0, rows_per_tile)],
                                 acc.at[pl.ds(base, rows_per_tile)], zsem)

        for p in range(phases):
            pltpu.sync_copy(src_hbm.at[wid, pl.ds(p * cpp, cpp)], idx_s)
            pltpu.sync_copy(dst_hbm.at[wid, pl.ds(p * cpp, cpp)], idx_d)
            for b in range(nbuf):
                pltpu.async_copy(table_hbm.at[idx_s.at[b]], rows.at[b],
                                 gsem.at[b])
            if p == 0:
                zcopy.wait()
                plsc.subcore_barrier()

            def outer(jj, carry):
                c0 = jj * nbuf
                for b in range(nbuf):
                    c = c0 + b
                    pltpu.make_async_copy(table_hbm.at[idx_s.at[c]],
                                          rows.at[b], gsem.at[b]).wait()
                    pltpu.async_copy(rows.at[b], acc.at[idx_d.at[c]],
                                     ssem.at[b], add=True)
                    pltpu.make_async_copy(rows.at[b], acc.at[idx_d.at[c]],
                                          ssem.at[b]).wait()

                    @pl.when(jj < n_outer - 1)
                    def _():
                        pltpu.async_copy(table_hbm.at[idx_s.at[c + nbuf]],
                                         rows.at[b], gsem.at[b])
                return carry

            lax.fori_loop(0, n_outer, outer, 0)
        plsc.subcore_barrier()

        pltpu.sync_copy(acc.at[pl.ds(base, rows_per_tile)],
                        out_hbm.at[cid, pl.ds(base, rows_per_tile)])

    return k


BLK = 400


def _mlp_body(scale_ref, h_ref, p_ref, w1_ref, b1_ref, w2_ref, b2_ref, o_ref):
    z = h_ref[...] * scale_ref[...] + p_ref[0] + p_ref[1]
    a = jnp.dot(z, w1_ref[...], preferred_element_type=jnp.float32) + b1_ref[...]
    a = jnp.maximum(a, 0.0)
    o = jnp.dot(a, w2_ref[...], preferred_element_type=jnp.float32) + b2_ref[...]
    o_ref[...] = jnp.maximum(o, 0.0)


def _mlp(scale, h, parts, w1f, b1f, w2f, b2f):
    return pl.pallas_call(
        _mlp_body,
        grid=(N // BLK,),
        in_specs=[
            pl.BlockSpec((1, H), lambda i: (0, 0)),
            pl.BlockSpec((BLK, D), lambda i: (i, 0)),
            pl.BlockSpec((NC, BLK, D), lambda i: (0, i, 0)),
            pl.BlockSpec((D, H), lambda i: (0, 0)),
            pl.BlockSpec((1, H), lambda i: (0, 0)),
            pl.BlockSpec((H, H), lambda i: (0, 0)),
            pl.BlockSpec((1, H), lambda i: (0, 0)),
        ],
        out_specs=pl.BlockSpec((BLK, H), lambda i: (i, 0)),
        out_shape=jax.ShapeDtypeStruct((N, H), jnp.float32),
    )(scale, h, parts, w1f, b1f, w2f, b2f)


def _head_body(h_ref, oh_ref, cw1_ref, cb1_ref, cw2_ref, cb2_ref, o_ref,
               acc_ref):
    i = pl.program_id(0)

    @pl.when(i == 0)
    def _():
        acc_ref[...] = jnp.zeros_like(acc_ref)

    acc_ref[...] += lax.dot_general(
        oh_ref[...], h_ref[...], (((0,), (0,)), ((), ())),
        preferred_element_type=jnp.float32)

    @pl.when(i == pl.num_programs(0) - 1)
    def _():
        p = acc_ref[...]
        a = jnp.dot(p, cw1_ref[...], preferred_element_type=jnp.float32) + cb1_ref[...]
        a = jnp.maximum(a, 0.0)
        o_ref[...] = jnp.dot(a, cw2_ref[...], preferred_element_type=jnp.float32) + cb2_ref[...]


def _head(h, onehot, cw1, cb1, cw2, cb2):
    return pl.pallas_call(
        _head_body,
        grid=(N // BLK,),
        in_specs=[
            pl.BlockSpec((BLK, D), lambda i: (i, 0)),
            pl.BlockSpec((BLK, G), lambda i: (i, 0)),
            pl.BlockSpec((H, H), lambda i: (0, 0)),
            pl.BlockSpec((1, H), lambda i: (0, 0)),
            pl.BlockSpec((H, OUT), lambda i: (0, 0)),
            pl.BlockSpec((1, OUT), lambda i: (0, 0)),
        ],
        out_specs=pl.BlockSpec((G, OUT), lambda i: (0, 0)),
        out_shape=jax.ShapeDtypeStruct((G, OUT), jnp.float32),
        scratch_shapes=[pltpu.VMEM((G, H), jnp.float32)],
    )(h, onehot, cw1, cb1, cw2, cb2)


def kernel(x, ei, batch, eps, w1, b1, g1, be1, w2, b2, g2, be2, cw1, cb1, cw2, cb2):
    s1 = g1 / jnp.sqrt(1.0 + 1e-5)
    w1f = w1 * s1[:, None, :]
    b1f = (b1 * s1 + be1)[:, None, :]
    s2 = g2 / jnp.sqrt(1.0 + 1e-5)
    w2f = w2 * s2[:, None, :]
    b2f = (b2 * s2 + be2)[:, None, :]

    pad_e = E_PAD - E
    src = jnp.concatenate(
        [ei[0], jnp.zeros((pad_e,), jnp.int32)]).reshape(NW, EDGE_CHUNKS, CHUNK)
    dst = jnp.concatenate(
        [ei[1], jnp.full((pad_e,), ACC_ROWS - 1, jnp.int32)]
    ).reshape(NW, EDGE_CHUNKS, CHUNK)

    onehot = (batch[:, None] == jnp.arange(G, dtype=jnp.int32)[None, :]
              ).astype(jnp.float32)

    zeros = jnp.zeros((ROWS_PER_TILE, D), jnp.float32)
    edge_scatter = _make_scatter_add(EDGE_CHUNKS, ACC_ROWS, phases=2)

    h = x
    for i in range(L):
        parts = edge_scatter(h, src, dst, zeros)
        scale = jnp.full((1, H), 1.0 + eps[i], jnp.float32)
        h = _mlp(scale, h, parts, w1f[i], b1f[i], w2f[i], b2f[i])

    return _head(h, onehot, cw1, cb1[None], cw2, cb2[None])
